# baseline (speedup 1.0000x reference)
import functools

import jax
import jax.numpy as jnp
from jax import lax
from jax.experimental import pallas as pl
from jax.experimental.pallas import tpu as pltpu
from jax.experimental.pallas import tpu_sc as plsc

N_NODES = 10000
N_EDGES = 160000
DIM = 256

NC = 2
NS = 16
NW = NC * NS
TILE_E = N_EDGES // NW
NVREG = TILE_E // 16
IDX_PAD = TILE_E + 16
CH = 96
NCHUNK = 52
TAIL = TILE_E - NCHUNK * CH
NBUF = 4
NROUND = NCHUNK // NBUF

_mesh = plsc.VectorSubcoreMesh(core_axis_name="c", subcore_axis_name="s")
_sc_params = pltpu.CompilerParams(needs_layout_passes=False)


def _node_scores_body(src_ref, dst_ref, w1_ref, w2_ref, ssrc_ref, sdst_ref):
    ssrc_ref[...] = jnp.sum(src_ref[...] * w1_ref[...], axis=1)
    sdst_ref[...] = jnp.sum(dst_ref[...] * w2_ref[...], axis=1)


_node_scores = pl.pallas_call(
    _node_scores_body,
    out_shape=[
        jax.ShapeDtypeStruct((N_NODES,), jnp.float32),
        jax.ShapeDtypeStruct((N_NODES,), jnp.float32),
    ],
)


@functools.partial(
    pl.kernel,
    mesh=_mesh,
    compiler_params=_sc_params,
    out_type=jax.ShapeDtypeStruct((N_EDGES,), jnp.float32),
    scratch_types=[
        pltpu.VMEM((N_NODES,), jnp.float32),
        pltpu.VMEM((N_NODES,), jnp.float32),
        pltpu.VMEM((IDX_PAD,), jnp.int32),
        pltpu.VMEM((IDX_PAD,), jnp.int32),
        pltpu.VMEM((IDX_PAD,), jnp.float32),
    ],
)
def _edge_scores(ssrc_hbm, sdst_hbm, esrc_hbm, edst_hbm, out_hbm,
                 ssrc_v, sdst_v, esrc_v, edst_v, sc_v):
    wid = lax.axis_index("s") * NC + lax.axis_index("c")
    base = wid * TILE_E
    pltpu.sync_copy(ssrc_hbm, ssrc_v)
    pltpu.sync_copy(sdst_hbm, sdst_v)
    pltpu.sync_copy(esrc_hbm.at[pl.ds(base, TILE_E)],
                    esrc_v.at[pl.ds(0, TILE_E)])
    pltpu.sync_copy(edst_hbm.at[pl.ds(base, TILE_E)],
                    edst_v.at[pl.ds(0, TILE_E)])

    def body(i, carry):
        sl = pl.ds(i * 16, 16)
        a = plsc.load_gather(ssrc_v, [esrc_v[sl]])
        d = plsc.load_gather(sdst_v, [edst_v[sl]])
        sc_v[sl] = a + d
        return carry

    lax.fori_loop(0, NVREG, body, 0)

    tl = pl.ds(NVREG * 16, 16)
    mask = lax.iota(jnp.int32, 16) < (TILE_E - NVREG * 16)
    i_s = jnp.where(mask, esrc_v[tl], 0)
    i_d = jnp.where(mask, edst_v[tl], 0)
    sc_v[tl] = plsc.load_gather(ssrc_v, [i_s]) + plsc.load_gather(sdst_v, [i_d])

    pltpu.sync_copy(sc_v.at[pl.ds(0, TILE_E)], out_hbm.at[pl.ds(base, TILE_E)])


def _softmax_body(sc_ref, out_ref):
    x = sc_ref[...]
    m = jnp.max(x)
    e = jnp.exp(x - m)
    out_ref[...] = e * (1.0 / jnp.sum(e))


_softmax = pl.pallas_call(
    _softmax_body,
    out_shape=jax.ShapeDtypeStruct((N_EDGES // 128, 128), jnp.float32),
)


@functools.partial(
    pl.kernel,
    mesh=_mesh,
    compiler_params=_sc_params,
    out_type=jax.ShapeDtypeStruct((N_EDGES, DIM), jnp.float32),
    scratch_types=[
        pltpu.VMEM((TILE_E,), jnp.int32),
        pltpu.VMEM((TILE_E,), jnp.float32),
        *[pltpu.VMEM((CH, DIM), jnp.float32) for _ in range(NBUF)],
        *[pltpu.SemaphoreType.DMA for _ in range(2 * NBUF)],
    ],
)
def _gather_scale(src_hbm, esrc_hbm, w_hbm, out_hbm, idx_v, wt_v, *bufs_sems):
    rows = bufs_sems[:NBUF]
    gsem = bufs_sems[NBUF:2 * NBUF]
    osem = bufs_sems[2 * NBUF:]
    wid = lax.axis_index("s") * NC + lax.axis_index("c")
    base = wid * TILE_E
    pltpu.sync_copy(esrc_hbm.at[pl.ds(base, TILE_E)], idx_v)
    pltpu.sync_copy(w_hbm.at[pl.ds(base, TILE_E)], wt_v)

    def gather(ci, b):
        return pltpu.make_async_copy(
            src_hbm.at[idx_v.at[pl.ds(ci * CH, CH)]], rows[b], gsem[b])

    def writeout(ci, b):
        return pltpu.make_async_copy(
            rows[b], out_hbm.at[pl.ds(base + ci * CH, CH)], osem[b])

    def scale(off, b):
        rb = rows[b]

        @plsc.parallel_loop(0, CH, unroll=2)
        def srow(e):
            w = plsc.load_gather(wt_v, [jnp.full((16,), off + e, jnp.int32)])
            for j in range(DIM // 16):
                sl = pl.ds(j * 16, 16)
                rb[e, sl] = rb[e, sl] * w

    def slot(ci, b, wait_prev, issue_next):
        gather(ci, b).wait()
        scale(ci * CH, b)
        writeout(ci, b).start()
        b2 = (b - 2) % NBUF
        if wait_prev:
            writeout(ci - 2, b2).wait()
        if issue_next:
            gather(ci + NBUF - 2, b2).start()

    for b in range(NBUF - 2):
        gather(b, b).start()

    for b in range(NBUF):
        slot(b, b, wait_prev=(b >= 2), issue_next=True)

    def round_body(g, carry):
        for b in range(NBUF):
            slot(g * NBUF + b, b, wait_prev=True, issue_next=True)
        return carry

    lax.fori_loop(1, NROUND - 1, round_body, 0)

    last = (NROUND - 1) * NBUF
    for b in range(NBUF):
        ci = last + b
        slot(ci, b, wait_prev=True, issue_next=(ci + NBUF - 2 < NCHUNK))
    writeout(NCHUNK - 2, (NCHUNK - 2) % NBUF).wait()
    writeout(NCHUNK - 1, (NCHUNK - 1) % NBUF).wait()

    tb = rows[0]
    toff = NCHUNK * CH
    pltpu.async_copy(
        src_hbm.at[idx_v.at[pl.ds(toff, TAIL)]],
        tb.at[pl.ds(0, TAIL)], gsem[0]).wait()

    @plsc.parallel_loop(0, TAIL, unroll=2)
    def trow(e):
        w = plsc.load_gather(wt_v, [jnp.full((16,), toff + e, jnp.int32)])
        for j in range(DIM // 16):
            sl = pl.ds(j * 16, 16)
            tb[e, sl] = tb[e, sl] * w

    pltpu.async_copy(
        tb.at[pl.ds(0, TAIL)],
        out_hbm.at[pl.ds(base + toff, TAIL)], osem[0]).wait()


@jax.jit
def kernel(src, dst, edge_index, W, b):
    del b
    edge_index = edge_index.astype(jnp.int32)
    w1 = W[:DIM, 0].reshape(1, DIM)
    w2 = W[DIM:, 0].reshape(1, DIM)
    s_src, s_dst = _node_scores(src, dst, w1, w2)
    scores = _edge_scores(s_src, s_dst, edge_index[0], edge_index[1])
    weights = _softmax(scores.reshape(N_EDGES // 128, 128)).reshape(-1)
    return _gather_scale(src, edge_index[0], weights)


# speedup vs baseline: 1.0124x; 1.0124x over previous
import functools

import jax
import jax.numpy as jnp
from jax import lax
from jax.experimental import pallas as pl
from jax.experimental.pallas import tpu as pltpu
from jax.experimental.pallas import tpu_sc as plsc

N_NODES = 10000
N_EDGES = 160000
DIM = 256

NC = 2
NS = 16
NW = NC * NS
TILE_E = N_EDGES // NW
NVREG = TILE_E // 16
IDX_PAD = TILE_E + 16
CH = 96
NCHUNK = 52
TAIL = TILE_E - NCHUNK * CH
NBUF = 4
NROUND = NCHUNK // NBUF

_mesh = plsc.VectorSubcoreMesh(core_axis_name="c", subcore_axis_name="s")
_sc_params = pltpu.CompilerParams(needs_layout_passes=False)


def _node_scores_body(src_ref, dst_ref, w1_ref, w2_ref, ssrc_ref, sdst_ref):
    ssrc_ref[...] = jnp.sum(src_ref[...] * w1_ref[...], axis=1)
    sdst_ref[...] = jnp.sum(dst_ref[...] * w2_ref[...], axis=1)


_node_scores = pl.pallas_call(
    _node_scores_body,
    out_shape=[
        jax.ShapeDtypeStruct((N_NODES,), jnp.float32),
        jax.ShapeDtypeStruct((N_NODES,), jnp.float32),
    ],
)


@functools.partial(
    pl.kernel,
    mesh=_mesh,
    compiler_params=_sc_params,
    out_type=jax.ShapeDtypeStruct((N_EDGES,), jnp.float32),
    scratch_types=[
        pltpu.VMEM((N_NODES,), jnp.float32),
        pltpu.VMEM((N_NODES,), jnp.float32),
        pltpu.VMEM((IDX_PAD,), jnp.int32),
        pltpu.VMEM((IDX_PAD,), jnp.int32),
        pltpu.VMEM((IDX_PAD,), jnp.float32),
        *[pltpu.SemaphoreType.DMA for _ in range(4)],
    ],
)
def _edge_scores(ssrc_hbm, sdst_hbm, esrc_hbm, edst_hbm, out_hbm,
                 ssrc_v, sdst_v, esrc_v, edst_v, sc_v, *sems):
    wid = lax.axis_index("s") * NC + lax.axis_index("c")
    base = wid * TILE_E
    cps = [
        pltpu.async_copy(ssrc_hbm, ssrc_v, sems[0]),
        pltpu.async_copy(sdst_hbm, sdst_v, sems[1]),
        pltpu.async_copy(esrc_hbm.at[pl.ds(base, TILE_E)],
                         esrc_v.at[pl.ds(0, TILE_E)], sems[2]),
        pltpu.async_copy(edst_hbm.at[pl.ds(base, TILE_E)],
                         edst_v.at[pl.ds(0, TILE_E)], sems[3]),
    ]
    for cp in cps:
        cp.wait()

    def body(i, carry):
        sl = pl.ds(i * 16, 16)
        a = plsc.load_gather(ssrc_v, [esrc_v[sl]])
        d = plsc.load_gather(sdst_v, [edst_v[sl]])
        sc_v[sl] = a + d
        return carry

    lax.fori_loop(0, NVREG, body, 0)

    tl = pl.ds(NVREG * 16, 16)
    mask = lax.iota(jnp.int32, 16) < (TILE_E - NVREG * 16)
    i_s = jnp.where(mask, esrc_v[tl], 0)
    i_d = jnp.where(mask, edst_v[tl], 0)
    sc_v[tl] = plsc.load_gather(ssrc_v, [i_s]) + plsc.load_gather(sdst_v, [i_d])

    pltpu.sync_copy(sc_v.at[pl.ds(0, TILE_E)], out_hbm.at[pl.ds(base, TILE_E)])


def _softmax_body(sc_ref, out_ref):
    x = sc_ref[...]
    m = jnp.max(x)
    e = jnp.exp(x - m)
    out_ref[...] = e * (1.0 / jnp.sum(e))


_softmax = pl.pallas_call(
    _softmax_body,
    out_shape=jax.ShapeDtypeStruct((N_EDGES // 128, 128), jnp.float32),
)


@functools.partial(
    pl.kernel,
    mesh=_mesh,
    compiler_params=_sc_params,
    out_type=jax.ShapeDtypeStruct((N_EDGES, DIM), jnp.float32),
    scratch_types=[
        pltpu.VMEM((TILE_E,), jnp.int32),
        pltpu.VMEM((TILE_E,), jnp.float32),
        *[pltpu.VMEM((CH, DIM), jnp.float32) for _ in range(NBUF)],
        *[pltpu.SemaphoreType.DMA for _ in range(2 * NBUF)],
    ],
)
def _gather_scale(src_hbm, esrc_hbm, w_hbm, out_hbm, idx_v, wt_v, *bufs_sems):
    rows = bufs_sems[:NBUF]
    gsem = bufs_sems[NBUF:2 * NBUF]
    osem = bufs_sems[2 * NBUF:]
    wid = lax.axis_index("s") * NC + lax.axis_index("c")
    base = wid * TILE_E
    pltpu.sync_copy(esrc_hbm.at[pl.ds(base, TILE_E)], idx_v)
    pltpu.sync_copy(w_hbm.at[pl.ds(base, TILE_E)], wt_v)

    def gather(ci, b):
        return pltpu.make_async_copy(
            src_hbm.at[idx_v.at[pl.ds(ci * CH, CH)]], rows[b], gsem[b])

    def writeout(ci, b):
        return pltpu.make_async_copy(
            rows[b], out_hbm.at[pl.ds(base + ci * CH, CH)], osem[b])

    def scale(off, b):
        rb = rows[b]

        @plsc.parallel_loop(0, CH, unroll=2)
        def srow(e):
            w = plsc.load_gather(wt_v, [jnp.full((16,), off + e, jnp.int32)])
            for j in range(DIM // 16):
                sl = pl.ds(j * 16, 16)
                rb[e, sl] = rb[e, sl] * w

    def slot(ci, b, wait_prev, issue_next):
        gather(ci, b).wait()
        scale(ci * CH, b)
        writeout(ci, b).start()
        b2 = (b - 2) % NBUF
        if wait_prev:
            writeout(ci - 2, b2).wait()
        if issue_next:
            gather(ci + NBUF - 2, b2).start()

    for b in range(NBUF - 2):
        gather(b, b).start()

    for b in range(NBUF):
        slot(b, b, wait_prev=(b >= 2), issue_next=True)

    def round_body(g, carry):
        for b in range(NBUF):
            slot(g * NBUF + b, b, wait_prev=True, issue_next=True)
        return carry

    lax.fori_loop(1, NROUND - 1, round_body, 0)

    last = (NROUND - 1) * NBUF
    for b in range(NBUF):
        ci = last + b
        slot(ci, b, wait_prev=True, issue_next=(ci + NBUF - 2 < NCHUNK))
    writeout(NCHUNK - 2, (NCHUNK - 2) % NBUF).wait()
    writeout(NCHUNK - 1, (NCHUNK - 1) % NBUF).wait()

    tb = rows[0]
    toff = NCHUNK * CH
    pltpu.async_copy(
        src_hbm.at[idx_v.at[pl.ds(toff, TAIL)]],
        tb.at[pl.ds(0, TAIL)], gsem[0]).wait()

    @plsc.parallel_loop(0, TAIL, unroll=2)
    def trow(e):
        w = plsc.load_gather(wt_v, [jnp.full((16,), toff + e, jnp.int32)])
        for j in range(DIM // 16):
            sl = pl.ds(j * 16, 16)
            tb[e, sl] = tb[e, sl] * w

    pltpu.async_copy(
        tb.at[pl.ds(0, TAIL)],
        out_hbm.at[pl.ds(base + toff, TAIL)], osem[0]).wait()


@jax.jit
def kernel(src, dst, edge_index, W, b):
    del b
    edge_index = edge_index.astype(jnp.int32)
    w1 = W[:DIM, 0].reshape(1, DIM)
    w2 = W[DIM:, 0].reshape(1, DIM)
    s_src, s_dst = _node_scores(src, dst, w1, w2)
    scores = _edge_scores(s_src, s_dst, edge_index[0], edge_index[1])
    weights = _softmax(scores.reshape(N_EDGES // 128, 128)).reshape(-1)
    return _gather_scale(src, edge_index[0], weights)


# speedup vs baseline: 1.0158x; 1.0033x over previous
import functools

import jax
import jax.numpy as jnp
from jax import lax
from jax.experimental import pallas as pl
from jax.experimental.pallas import tpu as pltpu
from jax.experimental.pallas import tpu_sc as plsc

N_NODES = 10000
N_EDGES = 160000
DIM = 256

NC = 2
NS = 16
NW = NC * NS
TILE_E = N_EDGES // NW
NVREG = TILE_E // 16
IDX_PAD = TILE_E + 16
CH = 96
NCHUNK = 52
TAIL = TILE_E - NCHUNK * CH
NBUF = 4
NROUND = NCHUNK // NBUF

_mesh = plsc.VectorSubcoreMesh(core_axis_name="c", subcore_axis_name="s")
_sc_params = pltpu.CompilerParams(needs_layout_passes=False)


def _node_scores_body(src_ref, dst_ref, w1_ref, w2_ref, ssrc_ref, sdst_ref):
    ssrc_ref[...] = jnp.sum(src_ref[...] * w1_ref[...], axis=1)
    sdst_ref[...] = jnp.sum(dst_ref[...] * w2_ref[...], axis=1)


_node_scores = pl.pallas_call(
    _node_scores_body,
    out_shape=[
        jax.ShapeDtypeStruct((N_NODES,), jnp.float32),
        jax.ShapeDtypeStruct((N_NODES,), jnp.float32),
    ],
)


@functools.partial(
    pl.kernel,
    mesh=_mesh,
    compiler_params=_sc_params,
    out_type=jax.ShapeDtypeStruct((N_EDGES,), jnp.float32),
    scratch_types=[
        pltpu.VMEM((N_NODES,), jnp.float32),
        pltpu.VMEM((N_NODES,), jnp.float32),
        pltpu.VMEM((IDX_PAD,), jnp.int32),
        pltpu.VMEM((IDX_PAD,), jnp.int32),
        pltpu.VMEM((IDX_PAD,), jnp.float32),
        *[pltpu.SemaphoreType.DMA for _ in range(4)],
    ],
)
def _edge_scores(ssrc_hbm, sdst_hbm, esrc_hbm, edst_hbm, out_hbm,
                 ssrc_v, sdst_v, esrc_v, edst_v, sc_v, *sems):
    wid = lax.axis_index("s") * NC + lax.axis_index("c")
    base = wid * TILE_E
    cps = [
        pltpu.async_copy(ssrc_hbm, ssrc_v, sems[0]),
        pltpu.async_copy(sdst_hbm, sdst_v, sems[1]),
        pltpu.async_copy(esrc_hbm.at[pl.ds(base, TILE_E)],
                         esrc_v.at[pl.ds(0, TILE_E)], sems[2]),
        pltpu.async_copy(edst_hbm.at[pl.ds(base, TILE_E)],
                         edst_v.at[pl.ds(0, TILE_E)], sems[3]),
    ]
    for cp in cps:
        cp.wait()

    def body(i, carry):
        sl = pl.ds(i * 16, 16)
        a = plsc.load_gather(ssrc_v, [esrc_v[sl]])
        d = plsc.load_gather(sdst_v, [edst_v[sl]])
        sc_v[sl] = a + d
        return carry

    lax.fori_loop(0, NVREG, body, 0)

    tl = pl.ds(NVREG * 16, 16)
    mask = lax.iota(jnp.int32, 16) < (TILE_E - NVREG * 16)
    i_s = jnp.where(mask, esrc_v[tl], 0)
    i_d = jnp.where(mask, edst_v[tl], 0)
    sc_v[tl] = plsc.load_gather(ssrc_v, [i_s]) + plsc.load_gather(sdst_v, [i_d])

    pltpu.sync_copy(sc_v.at[pl.ds(0, TILE_E)], out_hbm.at[pl.ds(base, TILE_E)])


def _softmax_body(sc_ref, out_ref):
    x = sc_ref[...]
    m = jnp.max(x)
    e = jnp.exp(x - m)
    out_ref[...] = e * (1.0 / jnp.sum(e))


_softmax = pl.pallas_call(
    _softmax_body,
    out_shape=jax.ShapeDtypeStruct((N_EDGES // 128, 128), jnp.float32),
)


@functools.partial(
    pl.kernel,
    mesh=_mesh,
    compiler_params=_sc_params,
    out_type=jax.ShapeDtypeStruct((N_EDGES, DIM), jnp.float32),
    scratch_types=[
        pltpu.VMEM((TILE_E,), jnp.int32),
        pltpu.VMEM((TILE_E,), jnp.float32),
        *[pltpu.VMEM((CH, DIM), jnp.float32) for _ in range(NBUF)],
        *[pltpu.SemaphoreType.DMA for _ in range(2 * NBUF)],
    ],
)
def _gather_scale(src_hbm, esrc_hbm, w_hbm, out_hbm, idx_v, wt_v, *bufs_sems):
    rows = bufs_sems[:NBUF]
    gsem = bufs_sems[NBUF:2 * NBUF]
    osem = bufs_sems[2 * NBUF:]
    wid = lax.axis_index("s") * NC + lax.axis_index("c")
    base = wid * TILE_E
    icp = pltpu.async_copy(esrc_hbm.at[pl.ds(base, TILE_E)], idx_v, gsem[-1])
    wcp = pltpu.async_copy(w_hbm.at[pl.ds(base, TILE_E)], wt_v, osem[-1])
    icp.wait()

    def gather(ci, b):
        return pltpu.make_async_copy(
            src_hbm.at[idx_v.at[pl.ds(ci * CH, CH)]], rows[b], gsem[b])

    def writeout(ci, b):
        return pltpu.make_async_copy(
            rows[b], out_hbm.at[pl.ds(base + ci * CH, CH)], osem[b])

    def scale(off, b):
        rb = rows[b]

        @plsc.parallel_loop(0, CH, unroll=2)
        def srow(e):
            w = plsc.load_gather(wt_v, [jnp.full((16,), off + e, jnp.int32)])
            for j in range(DIM // 16):
                sl = pl.ds(j * 16, 16)
                rb[e, sl] = rb[e, sl] * w

    def slot(ci, b, wait_prev, issue_next):
        gather(ci, b).wait()
        scale(ci * CH, b)
        writeout(ci, b).start()
        b2 = (b - 2) % NBUF
        if wait_prev:
            writeout(ci - 2, b2).wait()
        if issue_next:
            gather(ci + NBUF - 2, b2).start()

    for b in range(NBUF - 2):
        gather(b, b).start()
    wcp.wait()

    for b in range(NBUF):
        slot(b, b, wait_prev=(b >= 2), issue_next=True)

    def round_body(g, carry):
        for b in range(NBUF):
            slot(g * NBUF + b, b, wait_prev=True, issue_next=True)
        return carry

    lax.fori_loop(1, NROUND - 1, round_body, 0)

    last = (NROUND - 1) * NBUF
    for b in range(NBUF):
        ci = last + b
        slot(ci, b, wait_prev=True, issue_next=(ci + NBUF - 2 < NCHUNK))
    writeout(NCHUNK - 2, (NCHUNK - 2) % NBUF).wait()
    writeout(NCHUNK - 1, (NCHUNK - 1) % NBUF).wait()

    tb = rows[0]
    toff = NCHUNK * CH
    pltpu.async_copy(
        src_hbm.at[idx_v.at[pl.ds(toff, TAIL)]],
        tb.at[pl.ds(0, TAIL)], gsem[0]).wait()

    @plsc.parallel_loop(0, TAIL, unroll=2)
    def trow(e):
        w = plsc.load_gather(wt_v, [jnp.full((16,), toff + e, jnp.int32)])
        for j in range(DIM // 16):
            sl = pl.ds(j * 16, 16)
            tb[e, sl] = tb[e, sl] * w

    pltpu.async_copy(
        tb.at[pl.ds(0, TAIL)],
        out_hbm.at[pl.ds(base + toff, TAIL)], osem[0]).wait()


@jax.jit
def kernel(src, dst, edge_index, W, b):
    del b
    edge_index = edge_index.astype(jnp.int32)
    w1 = W[:DIM, 0].reshape(1, DIM)
    w2 = W[DIM:, 0].reshape(1, DIM)
    s_src, s_dst = _node_scores(src, dst, w1, w2)
    scores = _edge_scores(s_src, s_dst, edge_index[0], edge_index[1])
    weights = _softmax(scores.reshape(N_EDGES // 128, 128)).reshape(-1)
    return _gather_scale(src, edge_index[0], weights)
